# Initial kernel scaffold; baseline (speedup 1.0000x reference)
import jax
import jax.numpy as jnp
from jax.experimental import pallas as pl


def kernel(features, sigma):
    raise NotImplementedError("write your pallas kernel here")


# speedup vs baseline: 31.1919x; 31.1919x over previous
import jax
import jax.numpy as jnp
from jax.experimental import pallas as pl

_TEMPERATURE = 0.1
_SIGMA_STATIC = 5
_ROW_BLOCK = 256


def _cts_loss_kernel(fb_ref, f_ref, acc_ref, *, blk):
    i = pl.program_id(0)
    B = f_ref.shape[0]
    nsel = _SIGMA_STATIC + 1
    inv_t = 1.0 / _TEMPERATURE

    f = f_ref[...]
    norm = jnp.sqrt(jnp.sum(f * f, axis=1, keepdims=True))
    fn = f / jnp.maximum(norm, 1e-12)

    fb = fb_ref[...]
    bnorm = jnp.sqrt(jnp.sum(fb * fb, axis=1, keepdims=True))
    fblk = fb / jnp.maximum(bnorm, 1e-12)

    sim = jax.lax.dot_general(
        fblk, fn,
        dimension_numbers=(((1,), (1,)), ((), ())),
        preferred_element_type=jnp.float32,
    )

    esum = jnp.sum(jnp.exp(sim * inv_t), axis=1)

    iota = jax.lax.broadcasted_iota(jnp.int32, (blk, B), 1)
    work = sim
    top_sum = jnp.zeros((blk,), jnp.float32)
    exp_top = jnp.zeros((blk,), jnp.float32)
    for k in range(nsel):
        m = jnp.max(work, axis=1, keepdims=True)
        first = jnp.min(jnp.where(work == m, iota, B), axis=1, keepdims=True)
        work = jnp.where(iota == first, -3.0, work)
        if k >= 1:
            top_sum = top_sum + m[:, 0]
        exp_top = exp_top + jnp.exp(m[:, 0] * inv_t)

    neg_sum = esum - exp_top
    row_loss = top_sum * inv_t - _SIGMA_STATIC * jnp.log(neg_sum)
    partial = jnp.sum(row_loss).reshape(1, 1)

    @pl.when(i == 0)
    def _init():
        acc_ref[...] = jnp.zeros((1, 1), jnp.float32)

    acc_ref[...] += partial


def kernel(features, sigma):
    B, D = features.shape
    blk = min(_ROW_BLOCK, B)
    grid = (B // blk,)

    import functools
    acc = pl.pallas_call(
        functools.partial(_cts_loss_kernel, blk=blk),
        grid=grid,
        in_specs=[pl.BlockSpec((blk, D), lambda i: (i, 0)),
                  pl.BlockSpec((B, D), lambda i: (0, 0))],
        out_specs=pl.BlockSpec((1, 1), lambda i: (0, 0)),
        out_shape=jax.ShapeDtypeStruct((1, 1), jnp.float32),
    )(features, features)

    loss = -(acc[0, 0] / sigma) / B
    return jnp.maximum(loss, jnp.asarray(0.0, dtype=loss.dtype))


# speedup vs baseline: 38.1954x; 1.2245x over previous
import functools

import jax
import jax.numpy as jnp
from jax.experimental import pallas as pl
from jax.experimental.pallas import tpu as pltpu

_TEMPERATURE = 0.1
_SIGMA_STATIC = 5
_ROW_BLOCK = 256


def _cts_loss_kernel(f_ref, acc_ref, fn_ref, *, blk):
    i = pl.program_id(0)
    B = f_ref.shape[0]
    nsel = _SIGMA_STATIC + 1
    inv_t = 1.0 / _TEMPERATURE

    @pl.when(i == 0)
    def _normalize():
        f = f_ref[...]
        norm = jnp.sqrt(jnp.sum(f * f, axis=1, keepdims=True))
        fn_ref[...] = f / jnp.maximum(norm, 1e-12)

    fn = fn_ref[...]
    fblk = fn_ref[pl.ds(i * blk, blk), :]

    sim = jax.lax.dot_general(
        fblk, fn,
        dimension_numbers=(((1,), (1,)), ((), ())),
        preferred_element_type=jnp.float32,
    )

    esum = jnp.sum(jnp.exp(sim * inv_t), axis=1)

    work = sim
    top_sum = jnp.zeros((blk,), jnp.float32)
    exp_top = jnp.zeros((blk,), jnp.float32)
    taken = jnp.zeros((blk,), jnp.float32)
    for k in range(nsel):
        m = jnp.max(work, axis=1, keepdims=True)
        eq = work == m
        cnt = jnp.sum(eq.astype(jnp.float32), axis=1)
        use = jnp.minimum(cnt, nsel - taken)
        contrib = jnp.maximum(use - jnp.maximum(1.0 - taken, 0.0), 0.0)
        mv = m[:, 0]
        top_sum = top_sum + mv * contrib
        exp_top = exp_top + jnp.exp(mv * inv_t) * use
        taken = taken + use
        if k + 1 < nsel:
            work = jnp.where(eq, -3.0, work)

    neg_sum = esum - exp_top
    row_loss = top_sum * inv_t - _SIGMA_STATIC * jnp.log(neg_sum)
    partial = jnp.sum(row_loss).reshape(1, 1)

    @pl.when(i == 0)
    def _init():
        acc_ref[...] = jnp.zeros((1, 1), jnp.float32)

    acc_ref[...] += partial


def kernel(features, sigma):
    B, D = features.shape
    blk = min(_ROW_BLOCK, B)
    grid = (B // blk,)

    acc = pl.pallas_call(
        functools.partial(_cts_loss_kernel, blk=blk),
        grid=grid,
        in_specs=[pl.BlockSpec((B, D), lambda i: (0, 0))],
        out_specs=pl.BlockSpec((1, 1), lambda i: (0, 0)),
        out_shape=jax.ShapeDtypeStruct((1, 1), jnp.float32),
        scratch_shapes=[pltpu.VMEM((B, D), jnp.float32)],
    )(features)

    loss = -(acc[0, 0] / sigma) / B
    return jnp.maximum(loss, jnp.asarray(0.0, dtype=loss.dtype))


# speedup vs baseline: 38.7504x; 1.0145x over previous
import functools

import jax
import jax.numpy as jnp
from jax.experimental import pallas as pl
from jax.experimental.pallas import tpu as pltpu

_TEMPERATURE = 0.1
_SIGMA_STATIC = 5
_ROW_BLOCK = 256


def _cts_loss_kernel(f_ref, acc_ref, fn_ref, *, blk):
    i = pl.program_id(0)
    B = f_ref.shape[0]
    nsel = _SIGMA_STATIC + 1
    inv_t = 1.0 / _TEMPERATURE

    @pl.when(i == 0)
    def _normalize():
        f = f_ref[...]
        norm = jnp.sqrt(jnp.sum(f * f, axis=1, keepdims=True))
        fn_ref[...] = f / jnp.maximum(norm, 1e-12)

    fn = fn_ref[...]
    fblk = fn_ref[pl.ds(i * blk, blk), :]

    sim = jax.lax.dot_general(
        fblk, fn,
        dimension_numbers=(((1,), (1,)), ((), ())),
        preferred_element_type=jnp.float32,
    )

    ones = jnp.ones((B, 1), jnp.float32)

    def _rowsum(x):
        return jax.lax.dot_general(
            x, ones,
            dimension_numbers=(((1,), (0,)), ((), ())),
            preferred_element_type=jnp.float32,
        )

    esum = _rowsum(jnp.exp(sim * inv_t))

    work = sim
    top_sum = jnp.zeros((blk, 1), jnp.float32)
    exp_top = jnp.zeros((blk, 1), jnp.float32)
    taken = jnp.zeros((blk, 1), jnp.float32)
    for k in range(nsel):
        m = jnp.max(work, axis=1, keepdims=True)
        eq = work == m
        cnt = _rowsum(eq.astype(jnp.float32))
        use = jnp.minimum(cnt, nsel - taken)
        contrib = jnp.maximum(use - jnp.maximum(1.0 - taken, 0.0), 0.0)
        top_sum = top_sum + m * contrib
        exp_top = exp_top + jnp.exp(m * inv_t) * use
        taken = taken + use
        if k + 1 < nsel:
            work = jnp.where(eq, -3.0, work)

    neg_sum = esum - exp_top
    row_loss = top_sum * inv_t - _SIGMA_STATIC * jnp.log(neg_sum)
    partial = jnp.sum(row_loss).reshape(1, 1)

    @pl.when(i == 0)
    def _init():
        acc_ref[...] = jnp.zeros((1, 1), jnp.float32)

    acc_ref[...] += partial


def kernel(features, sigma):
    B, D = features.shape
    blk = min(_ROW_BLOCK, B)
    grid = (B // blk,)

    acc = pl.pallas_call(
        functools.partial(_cts_loss_kernel, blk=blk),
        grid=grid,
        in_specs=[pl.BlockSpec((B, D), lambda i: (0, 0))],
        out_specs=pl.BlockSpec((1, 1), lambda i: (0, 0)),
        out_shape=jax.ShapeDtypeStruct((1, 1), jnp.float32),
        scratch_shapes=[pltpu.VMEM((B, D), jnp.float32)],
    )(features)

    loss = -(acc[0, 0] / sigma) / B
    return jnp.maximum(loss, jnp.asarray(0.0, dtype=loss.dtype))
